# baseline (speedup 1.0000x reference)
import math

import jax
import jax.numpy as jnp
import numpy as np
from jax import lax
from jax.experimental import pallas as pl
from jax.experimental.pallas import tpu as pltpu
from jax.experimental.pallas import tpu_sc as plsc

MAXLEN = 512
NUM_CORES = 2
NUM_SUBCORES = 16
NW = NUM_CORES * NUM_SUBCORES
LANES = 16
CV = 256


def _make_pe_np(emb: int) -> np.ndarray:
    pe = np.zeros((MAXLEN, emb), dtype=np.float32)
    position = np.arange(0, MAXLEN, dtype=np.float32)[:, None]
    div_term = np.exp(
        np.arange(0, emb, 2, dtype=np.float32) * -(math.log(10000.0) / emb)
    )
    pe[:, 0::2] = np.sin(position * div_term)
    pe[:, 1::2] = np.cos(position * div_term)
    return pe


def _mesh():
    return plsc.VectorSubcoreMesh(
        core_axis_name="c",
        subcore_axis_name="s",
        num_cores=NUM_CORES,
        num_subcores=NUM_SUBCORES,
    )


def _reformat_table(w_t, w_tail, V, D):
    vfull = (V // 128) * 128
    nch = vfull // CV
    nround = (nch + NW - 1) // NW
    vtail = V - vfull

    @pl.kernel(
        out_type=jax.ShapeDtypeStruct((V * D,), jnp.float32),
        mesh=_mesh(),
        compiler_params=pltpu.CompilerParams(
            use_tc_tiling_on_sc=True, needs_layout_passes=False
        ),
        scratch_types=[
            pltpu.VMEM((3 * D, CV), jnp.float32),
            pltpu.VMEM((3 * CV * D,), jnp.float32),
            pltpu.VMEM((V - (V // 128) * 128, D), jnp.float32),
            pltpu.SemaphoreType.DMA,
            pltpu.SemaphoreType.DMA,
            pltpu.SemaphoreType.DMA,
            pltpu.SemaphoreType.DMA,
            pltpu.SemaphoreType.DMA,
            pltpu.SemaphoreType.DMA,
        ],
    )
    def body(wt_hbm, tail_hbm, out_hbm, src, tsl, tailb,
             lsem0, lsem1, lsem2, ssem0, ssem1, ssem2):
        wid = lax.axis_index("s") * NUM_CORES + lax.axis_index("c")
        lsems = (lsem0, lsem1, lsem2)
        ssems = (ssem0, ssem1, ssem2)
        eiotas = [lax.iota(jnp.int32, LANES) + j * LANES for j in range(D // LANES)]

        def chunk_id(c):
            return c * NW + wid

        def load_start(c, par):
            pltpu.make_async_copy(
                wt_hbm.at[:, pl.ds(chunk_id(c) * CV, CV)],
                src.at[pl.ds(par * D, D), :],
                lsems[par]
            ).start()

        def load_wait(c, par):
            pltpu.make_async_copy(
                wt_hbm.at[:, pl.ds(chunk_id(c) * CV, CV)],
                src.at[pl.ds(par * D, D), :],
                lsems[par]
            ).wait()

        def store_start(c, par):
            pltpu.make_async_copy(
                tsl.at[pl.ds(par * CV * D, CV * D)],
                out_hbm.at[pl.ds(chunk_id(c) * CV * D, CV * D)],
                ssems[par]
            ).start()

        def store_wait(c, par):
            pltpu.make_async_copy(
                tsl.at[pl.ds(par * CV * D, CV * D)],
                out_hbm.at[pl.ds(chunk_id(c) * CV * D, CV * D)],
                ssems[par]
            ).wait()

        def compute(par, nv):
            srcp = src.at[pl.ds(par * D, D), :]
            tslp = tsl.at[pl.ds(par * CV * D, CV * D)]

            @plsc.parallel_loop(0, nv, unroll=4)
            def _(v):
                vs = jnp.full((LANES,), v, jnp.int32)
                for j in range(D // LANES):
                    tslp[pl.ds(v * D + j * LANES, LANES)] = plsc.load_gather(
                        srcp, [eiotas[j], vs]
                    )

        for i in range(2):
            @pl.when(chunk_id(i) < nch)
            def _():
                load_start(i, i % 3)

        def step(c, par):
            @pl.when(chunk_id(c) < nch)
            def _():
                @pl.when(chunk_id(c + 2) < nch)
                def _():
                    load_start(c + 2, (par + 2) % 3)

                load_wait(c, par)

                @pl.when(c >= 3)
                def _():
                    store_wait(c - 3, par)

                compute(par, CV)
                store_start(c, par)

        def loop3(c3, _):
            step(3 * c3, 0)
            step(3 * c3 + 1, 1)
            step(3 * c3 + 2, 2)
            return 0

        lax.fori_loop(0, (nround + 2) // 3, loop3, 0)

        for cc in range(nround - 4, nround):
            @pl.when(
                jnp.logical_and(
                    chunk_id(cc) < nch, chunk_id(cc + 3) >= nch
                )
            )
            def _():
                store_wait(cc, cc % 3)

        @pl.when(wid == NW - 1)
        def _():
            pltpu.sync_copy(tail_hbm, tailb)
            tslp = tsl.at[pl.ds(0, CV * D)]

            @plsc.parallel_loop(0, vtail, unroll=4)
            def _(v):
                vs = jnp.full((LANES,), v, jnp.int32)
                for j in range(D // LANES):
                    tslp[pl.ds(v * D + j * LANES, LANES)] = plsc.load_gather(
                        tailb, [vs, eiotas[j]]
                    )

            pltpu.sync_copy(
                tsl.at[pl.ds(0, vtail * D)],
                out_hbm.at[pl.ds(vfull * D, vtail * D)],
            )

    return body(w_t, w_tail)


def _lookup(idx_t, w_flat, pe_arr, B, L, V, D):
    factor = math.sqrt(D)
    bpw = B // NW
    ng = bpw // LANES

    @pl.kernel(
        out_type=jax.ShapeDtypeStruct((L, D // 8, B // bpw, 8, bpw),
                                      jnp.float32),
        mesh=_mesh(),
        compiler_params=pltpu.CompilerParams(
            use_tc_tiling_on_sc=False, needs_layout_passes=False
        ),
        scratch_types=[
            pltpu.VMEM((L, bpw), jnp.int32),
            pltpu.VMEM((L * D,), jnp.float32),
            pltpu.VMEM((4, bpw, D), jnp.float32),
            pltpu.VMEM((4, D // 8, 8, bpw), jnp.float32),
            pltpu.SemaphoreType.DMA,
            pltpu.SemaphoreType.DMA,
            pltpu.SemaphoreType.DMA,
            pltpu.SemaphoreType.DMA,
            pltpu.SemaphoreType.DMA,
            pltpu.SemaphoreType.DMA,
            pltpu.SemaphoreType.DMA,
            pltpu.SemaphoreType.DMA,
        ],
    )
    def body(idx_hbm, w_hbm, pe_hbm, out_hbm, idx_v, pe_v, gb, sl,
             gsem0, gsem1, gsem2, gsem3, ssem0, ssem1, ssem2, ssem3):
        wid = lax.axis_index("s") * NUM_CORES + lax.axis_index("c")
        b0 = wid * bpw
        pltpu.sync_copy(idx_hbm.at[:, pl.ds(b0, bpw)], idx_v)
        pltpu.sync_copy(pe_hbm, pe_v)

        gsems = (gsem0, gsem1, gsem2, gsem3)
        ssems = (ssem0, ssem1, ssem2, ssem3)
        biotas = [lax.iota(jnp.int32, LANES) + k * LANES for k in range(ng)]

        def gather_start(l, par):
            pltpu.make_async_copy(
                w_hbm.at[idx_v.at[l]], gb.at[par], gsems[par]
            ).start()

        def gather_wait(l, par):
            pltpu.make_async_copy(
                w_hbm.at[idx_v.at[l]], gb.at[par], gsems[par]
            ).wait()

        def scatter_start(l, par):
            pltpu.make_async_copy(
                sl.at[par], out_hbm.at[l, :, wid], ssems[par]
            ).start()

        def scatter_wait(l, par):
            pltpu.make_async_copy(
                sl.at[par], out_hbm.at[l, :, wid], ssems[par]
            ).wait()

        for i in range(3):
            gather_start(i, i)

        def step(l, par):
            @pl.when(l + 3 < L)
            def _():
                gather_start(l + 3, (par + 3) % 4)

            gather_wait(l, par)

            @pl.when(l >= 4)
            def _():
                scatter_wait(l - 4, par)

            l64 = l * D
            gbp = gb.at[par]
            slp = sl.at[par]

            @plsc.parallel_loop(0, D, unroll=4)
            def _(e):
                pev = plsc.load_gather(
                    pe_v, [jnp.full((LANES,), l64 + e, jnp.int32)]
                )
                es = jnp.full((LANES,), e, jnp.int32)
                ehi = lax.shift_right_logical(e, 3)
                elo = lax.bitwise_and(e, 7)
                for k in range(ng):
                    v = plsc.load_gather(gbp, [biotas[k], es])
                    v = v * factor + pev
                    slp[ehi, elo, pl.ds(k * LANES, LANES)] = v

            scatter_start(l, par)

        def loop4(l4, _):
            for j in range(4):
                step(4 * l4 + j, j)
            return 0

        lax.fori_loop(0, L // 4, loop4, 0)
        for l in range(L - 4, L):
            scatter_wait(l, l % 4)

    return body(idx_t, w_flat, pe_arr)


def kernel(input, weight):
    B, L = input.shape
    V, D = weight.shape
    pe = jnp.asarray(_make_pe_np(D)[:L])
    vfull = (V // 128) * 128

    @jax.jit
    def run(inp, w, pe_arr):
        w_flat = _reformat_table(w.T, w[vfull:, :], V, D)
        out6 = _lookup(
            inp.T, w_flat.reshape(V, D), pe_arr.reshape(-1), B, L, V, D
        )
        return out6.transpose(2, 4, 0, 1, 3).reshape(B, L, D)

    return run(input, weight, pe)


# speedup vs baseline: 1.1530x; 1.1530x over previous
import math

import jax
import jax.numpy as jnp
import numpy as np
from jax import lax
from jax.experimental import pallas as pl
from jax.experimental.pallas import tpu as pltpu
from jax.experimental.pallas import tpu_sc as plsc

MAXLEN = 512
NUM_CORES = 2
NUM_SUBCORES = 16
NW = NUM_CORES * NUM_SUBCORES
LANES = 16
CV = 256


def _make_pe_np(emb: int) -> np.ndarray:
    pe = np.zeros((MAXLEN, emb), dtype=np.float32)
    position = np.arange(0, MAXLEN, dtype=np.float32)[:, None]
    div_term = np.exp(
        np.arange(0, emb, 2, dtype=np.float32) * -(math.log(10000.0) / emb)
    )
    pe[:, 0::2] = np.sin(position * div_term)
    pe[:, 1::2] = np.cos(position * div_term)
    return pe


def _mesh():
    return plsc.VectorSubcoreMesh(
        core_axis_name="c",
        subcore_axis_name="s",
        num_cores=NUM_CORES,
        num_subcores=NUM_SUBCORES,
    )


def _reformat_table(w_t, w_tail, V, D):
    vfull = (V // 128) * 128
    nch = vfull // CV
    nround = (nch + NW - 1) // NW
    vtail = V - vfull

    @pl.kernel(
        out_type=jax.ShapeDtypeStruct((V * D,), jnp.float32),
        mesh=_mesh(),
        compiler_params=pltpu.CompilerParams(
            use_tc_tiling_on_sc=True, needs_layout_passes=False
        ),
        scratch_types=[
            pltpu.VMEM((3 * D, CV), jnp.float32),
            pltpu.VMEM((3 * CV * D,), jnp.float32),
            pltpu.VMEM((V - (V // 128) * 128, D), jnp.float32),
            pltpu.SemaphoreType.DMA,
            pltpu.SemaphoreType.DMA,
            pltpu.SemaphoreType.DMA,
            pltpu.SemaphoreType.DMA,
            pltpu.SemaphoreType.DMA,
            pltpu.SemaphoreType.DMA,
        ],
    )
    def body(wt_hbm, tail_hbm, out_hbm, src, tsl, tailb,
             lsem0, lsem1, lsem2, ssem0, ssem1, ssem2):
        wid = lax.axis_index("s") * NUM_CORES + lax.axis_index("c")
        lsems = (lsem0, lsem1, lsem2)
        ssems = (ssem0, ssem1, ssem2)
        eiotas = [lax.iota(jnp.int32, LANES) + j * LANES for j in range(D // LANES)]

        def chunk_id(c):
            return c * NW + wid

        def load_start(c, par):
            pltpu.make_async_copy(
                wt_hbm.at[:, pl.ds(chunk_id(c) * CV, CV)],
                src.at[pl.ds(par * D, D), :],
                lsems[par]
            ).start()

        def load_wait(c, par):
            pltpu.make_async_copy(
                wt_hbm.at[:, pl.ds(chunk_id(c) * CV, CV)],
                src.at[pl.ds(par * D, D), :],
                lsems[par]
            ).wait()

        def store_start(c, par):
            pltpu.make_async_copy(
                tsl.at[pl.ds(par * CV * D, CV * D)],
                out_hbm.at[pl.ds(chunk_id(c) * CV * D, CV * D)],
                ssems[par]
            ).start()

        def store_wait(c, par):
            pltpu.make_async_copy(
                tsl.at[pl.ds(par * CV * D, CV * D)],
                out_hbm.at[pl.ds(chunk_id(c) * CV * D, CV * D)],
                ssems[par]
            ).wait()

        def compute(par, nv):
            srcp = src.at[pl.ds(par * D, D), :]
            tslp = tsl.at[pl.ds(par * CV * D, CV * D)]

            @plsc.parallel_loop(0, nv, unroll=4)
            def _(v):
                vs = jnp.full((LANES,), v, jnp.int32)
                for j in range(D // LANES):
                    tslp[pl.ds(v * D + j * LANES, LANES)] = plsc.load_gather(
                        srcp, [eiotas[j], vs]
                    )

        for i in range(2):
            @pl.when(chunk_id(i) < nch)
            def _():
                load_start(i, i % 3)

        def step(c, par):
            @pl.when(chunk_id(c) < nch)
            def _():
                @pl.when(chunk_id(c + 2) < nch)
                def _():
                    load_start(c + 2, (par + 2) % 3)

                load_wait(c, par)

                @pl.when(c >= 3)
                def _():
                    store_wait(c - 3, par)

                compute(par, CV)
                store_start(c, par)

        def loop3(c3, _):
            step(3 * c3, 0)
            step(3 * c3 + 1, 1)
            step(3 * c3 + 2, 2)
            return 0

        lax.fori_loop(0, (nround + 2) // 3, loop3, 0)

        for cc in range(nround - 4, nround):
            @pl.when(
                jnp.logical_and(
                    chunk_id(cc) < nch, chunk_id(cc + 3) >= nch
                )
            )
            def _():
                store_wait(cc, cc % 3)

        @pl.when(wid == NW - 1)
        def _():
            pltpu.sync_copy(tail_hbm, tailb)
            tslp = tsl.at[pl.ds(0, CV * D)]

            @plsc.parallel_loop(0, vtail, unroll=4)
            def _(v):
                vs = jnp.full((LANES,), v, jnp.int32)
                for j in range(D // LANES):
                    tslp[pl.ds(v * D + j * LANES, LANES)] = plsc.load_gather(
                        tailb, [vs, eiotas[j]]
                    )

            pltpu.sync_copy(
                tsl.at[pl.ds(0, vtail * D)],
                out_hbm.at[pl.ds(vfull * D, vtail * D)],
            )

    return body(w_t, w_tail)


def _lookup(idx_t, w_flat, pe_arr, B, L, V, D):
    factor = math.sqrt(D)
    bpw = B // NW
    ng = bpw // LANES

    @pl.kernel(
        out_type=jax.ShapeDtypeStruct((L, D // 8, B // bpw, 8, bpw),
                                      jnp.float32),
        mesh=_mesh(),
        compiler_params=pltpu.CompilerParams(
            use_tc_tiling_on_sc=False, needs_layout_passes=False
        ),
        scratch_types=[
            pltpu.VMEM((L, bpw), jnp.int32),
            pltpu.VMEM((L * D,), jnp.float32),
            pltpu.VMEM((4, bpw, D), jnp.float32),
            pltpu.VMEM((4, D // 8, 8, bpw), jnp.float32),
            pltpu.SemaphoreType.DMA,
            pltpu.SemaphoreType.DMA,
            pltpu.SemaphoreType.DMA,
            pltpu.SemaphoreType.DMA,
            pltpu.SemaphoreType.DMA,
            pltpu.SemaphoreType.DMA,
            pltpu.SemaphoreType.DMA,
            pltpu.SemaphoreType.DMA,
        ],
    )
    def body(idx_hbm, w_hbm, pe_hbm, out_hbm, idx_v, pe_v, gb, sl,
             gsem0, gsem1, gsem2, gsem3, ssem0, ssem1, ssem2, ssem3):
        wid = lax.axis_index("s") * NUM_CORES + lax.axis_index("c")
        b0 = wid * bpw
        pltpu.sync_copy(idx_hbm.at[:, pl.ds(b0, bpw)], idx_v)
        pltpu.sync_copy(pe_hbm, pe_v)

        gsems = (gsem0, gsem1, gsem2, gsem3)
        ssems = (ssem0, ssem1, ssem2, ssem3)
        biotas = [lax.iota(jnp.int32, LANES) + k * LANES for k in range(ng)]

        def gather_start(l, par):
            pltpu.make_async_copy(
                w_hbm.at[idx_v.at[l]], gb.at[par], gsems[par]
            ).start()

        def gather_wait(l, par):
            pltpu.make_async_copy(
                w_hbm.at[idx_v.at[l]], gb.at[par], gsems[par]
            ).wait()

        def scatter_start(l, par):
            pltpu.make_async_copy(
                sl.at[par], out_hbm.at[l, :, wid], ssems[par]
            ).start()

        def scatter_wait(l, par):
            pltpu.make_async_copy(
                sl.at[par], out_hbm.at[l, :, wid], ssems[par]
            ).wait()

        for i in range(3):
            gather_start(i, i)

        def step(l, par):
            @pl.when(l + 3 < L)
            def _():
                gather_start(l + 3, (par + 3) % 4)

            gather_wait(l, par)

            @pl.when(l >= 4)
            def _():
                scatter_wait(l - 4, par)

            l64 = l * D
            gbp = gb.at[par]
            slp = sl.at[par]

            @plsc.parallel_loop(0, D, unroll=4)
            def _(e):
                pev = plsc.load_gather(
                    pe_v, [jnp.full((LANES,), l64 + e, jnp.int32)]
                )
                es = jnp.full((LANES,), e, jnp.int32)
                ehi = lax.shift_right_logical(e, 3)
                elo = lax.bitwise_and(e, 7)
                for k in range(ng):
                    v = plsc.load_gather(gbp, [biotas[k], es])
                    v = v * factor + pev
                    slp[ehi, elo, pl.ds(k * LANES, LANES)] = v

            scatter_start(l, par)

        def loop4(l4, _):
            for j in range(4):
                step(4 * l4 + j, j)
            return 0

        lax.fori_loop(0, L // 4, loop4, 0)
        for l in range(L - 4, L):
            scatter_wait(l, l % 4)

    return body(idx_t, w_flat, pe_arr)


def kernel(input, weight):
    B, L = input.shape
    V, D = weight.shape
    pe = jnp.asarray(_make_pe_np(D)[:L])
    vfull = (V // 128) * 128

    @jax.jit
    def run(inp, w, pe_arr):
        out6 = _lookup(inp.T, w, pe_arr.reshape(-1), B, L, V, D)
        return out6.transpose(2, 4, 0, 1, 3).reshape(B, L, D)

    return run(input, weight, pe)
